# baseline (speedup 1.0000x reference)
import functools

import jax
import jax.numpy as jnp
from jax import lax
from jax.experimental import pallas as pl
from jax.experimental.pallas import tpu as pltpu
from jax.experimental.pallas import tpu_sc as plsc

_NC = 2
_NS = 16
_NW = _NC * _NS
_LANES = 16
_BLK = 128
_CH = 16


def _sc_aggregate(x, col_p, row_p, val_p, zeros, b_w, n_pad):
    _, d = x.shape
    rows_per_sub = n_pad // _NS
    mesh = plsc.VectorSubcoreMesh(core_axis_name="c", subcore_axis_name="s")

    @functools.partial(
        pl.kernel,
        out_type=jax.ShapeDtypeStruct((_NC, n_pad, d), jnp.float32),
        mesh=mesh,
        scratch_types=[
            pltpu.VMEM((_CH, _BLK), jnp.int32),
            pltpu.VMEM((_CH, _BLK), jnp.int32),
            pltpu.VMEM((_CH, _BLK), jnp.float32),
            pltpu.VMEM((_BLK, d), jnp.float32),
            pltpu.VMEM((_BLK, d), jnp.float32),
            pltpu.VMEM_SHARED((n_pad, d), jnp.float32),
            pltpu.SemaphoreType.DMA,
            pltpu.SemaphoreType.DMA,
            pltpu.SemaphoreType.DMA,
            pltpu.SemaphoreType.DMA,
        ],
    )
    def body(x_hbm, col_hbm, row_hbm, val_hbm, z_hbm, out_hbm,
             col_v, row_v, val_v, rows0_v, rows1_v, acc_sh,
             gsem0, gsem1, ssem0, ssem1):
        rows = (rows0_v, rows1_v)
        gsem = (gsem0, gsem1)
        ssem = (ssem0, ssem1)
        cid = lax.axis_index("c")
        sid = lax.axis_index("s")
        wid = sid * _NC + cid
        sub_rows = pl.ds(sid * rows_per_sub, rows_per_sub)

        pltpu.sync_copy(z_hbm, acc_sh.at[sub_rows])
        plsc.subcore_barrier()

        def gather(j, b):
            pltpu.async_copy(x_hbm.at[col_v.at[j]], rows[b], gsem[b])

        def gather_wait(j, b):
            pltpu.make_async_copy(x_hbm.at[col_v.at[j]], rows[b], gsem[b]).wait()

        def scatter(j, b):
            pltpu.async_copy(rows[b], acc_sh.at[row_v.at[j]], ssem[b], add=True)

        def scatter_wait(j, b):
            pltpu.make_async_copy(rows[b], acc_sh.at[row_v.at[j]], ssem[b]).wait()

        def chunk(ci, carry):
            cs = pl.ds(ci * _CH, _CH)
            pltpu.sync_copy(col_hbm.at[wid, cs], col_v)
            pltpu.sync_copy(row_hbm.at[wid, cs], row_v)
            pltpu.sync_copy(val_hbm.at[wid, cs], val_v)

            def process(j, c2):
                gather(j, 0)
                gather_wait(j, 0)

                def scale16(g, c3):
                    vv = val_v[j, pl.ds(g * _LANES, _LANES)]
                    base = g * _LANES
                    for k in range(_LANES):
                        v = vv[k]
                        for c in range(d // _LANES):
                            sl = pl.ds(c * _LANES, _LANES)
                            rows[0][base + k, sl] = rows[0][base + k, sl] * v
                    return c3

                lax.fori_loop(0, _BLK // _LANES, scale16, 0)
                scatter(j, 0)
                scatter_wait(j, 0)
                return c2

            lax.fori_loop(0, _CH, process, 0)
            return carry

        lax.fori_loop(0, b_w // _CH, chunk, 0)

        plsc.subcore_barrier()
        pltpu.sync_copy(acc_sh.at[sub_rows], out_hbm.at[cid].at[sub_rows])

    return body(x, col_p, row_p, val_p, zeros)


def _mm_body(p0_ref, p1_ref, w_ref, b_ref, o_ref):
    s = p0_ref[...] + p1_ref[...]
    acc = jnp.dot(s, w_ref[...], preferred_element_type=jnp.float32)
    o_ref[...] = jnp.maximum(acc + b_ref[...], 0.0)


def _tc_finish(partial, W, b, n):
    d_in = partial.shape[2]
    d_out = W.shape[1]
    bm = 1000 if n % 1000 == 0 else n
    return pl.pallas_call(
        _mm_body,
        grid=(n // bm,),
        in_specs=[
            pl.BlockSpec((bm, d_in), lambda i: (i, 0)),
            pl.BlockSpec((bm, d_in), lambda i: (i, 0)),
            pl.BlockSpec((d_in, d_out), lambda i: (0, 0)),
            pl.BlockSpec((1, d_out), lambda i: (0, 0)),
        ],
        out_specs=pl.BlockSpec((bm, d_out), lambda i: (i, 0)),
        out_shape=jax.ShapeDtypeStruct((n, d_out), jnp.float32),
    )(partial[0], partial[1], W, b.reshape(1, d_out))


def kernel(x, adj_indices, adj_values, W, b):
    n, d = x.shape
    e = adj_values.shape[0]
    n_blocks = pl.cdiv(e, _BLK)
    b_w = pl.cdiv(n_blocks, _NW)
    b_w = pl.cdiv(b_w, _CH) * _CH
    pad = _NW * b_w * _BLK - e
    col_p = jnp.pad(adj_indices[1], (0, pad)).reshape(_NW, b_w, _BLK)
    row_p = jnp.pad(adj_indices[0], (0, pad)).reshape(_NW, b_w, _BLK)
    val_p = jnp.pad(adj_values, (0, pad)).reshape(_NW, b_w, _BLK)
    n_pad = ((n + 8 * _NS - 1) // (8 * _NS)) * (8 * _NS)
    zeros = jnp.zeros((n_pad // _NS, d), jnp.float32)
    partial = _sc_aggregate(x, col_p, row_p, val_p, zeros, b_w, n_pad)
    return _tc_finish(partial, W, b, n)


# speedup vs baseline: 1.5023x; 1.5023x over previous
import functools

import jax
import jax.numpy as jnp
from jax import lax
from jax.experimental import pallas as pl
from jax.experimental.pallas import tpu as pltpu
from jax.experimental.pallas import tpu_sc as plsc

_NC = 2
_NS = 16
_NW = _NC * _NS
_LANES = 16
_BLK = 128
_CH = 16


def _sc_aggregate(x, col_p, row_p, val_p, zeros, b_w, n_pad):
    _, d = x.shape
    rows_per_sub = n_pad // _NS
    mesh = plsc.VectorSubcoreMesh(core_axis_name="c", subcore_axis_name="s")

    @functools.partial(
        pl.kernel,
        out_type=jax.ShapeDtypeStruct((_NC, n_pad, d), jnp.float32),
        mesh=mesh,
        scratch_types=[
            pltpu.VMEM((b_w, _BLK), jnp.int32),
            pltpu.VMEM((b_w, _BLK), jnp.int32),
            pltpu.VMEM((b_w, _BLK), jnp.float32),
            pltpu.VMEM((_BLK, d), jnp.float32),
            pltpu.VMEM_SHARED((n_pad, d), jnp.float32),
            pltpu.SemaphoreType.DMA,
        ],
    )
    def body(x_hbm, col_hbm, row_hbm, val_hbm, z_hbm, out_hbm,
             col_v, row_v, val_v, rows0_v, acc_sh, gsem0):
        rows = (rows0_v,)
        gsem = (gsem0,)
        cid = lax.axis_index("c")
        sid = lax.axis_index("s")
        wid = sid * _NC + cid
        sub_rows = pl.ds(sid * rows_per_sub, rows_per_sub)

        pltpu.sync_copy(z_hbm, acc_sh.at[sub_rows])
        pltpu.sync_copy(col_hbm.at[wid], col_v)
        pltpu.sync_copy(row_hbm.at[wid], row_v)
        pltpu.sync_copy(val_hbm.at[wid], val_v)
        plsc.subcore_barrier()

        def process_block(j, carry):
            pltpu.async_copy(x_hbm.at[col_v.at[j]], rows[0], gsem[0]).wait()

            def scale16(g, c2):
                vv = val_v[j, pl.ds(g * _LANES, _LANES)]
                base = g * _LANES
                for k in range(_LANES):
                    v = vv[k]
                    for c in range(d // _LANES):
                        sl = pl.ds(c * _LANES, _LANES)
                        rows[0][base + k, sl] = rows[0][base + k, sl] * v
                return c2

            lax.fori_loop(0, _BLK // _LANES, scale16, 0)
            pltpu.sync_copy(rows[0], acc_sh.at[row_v.at[j]], add=True)
            return carry

        lax.fori_loop(0, b_w, process_block, 0)

        plsc.subcore_barrier()
        pltpu.sync_copy(acc_sh.at[sub_rows], out_hbm.at[cid].at[sub_rows])

    return body(x, col_p, row_p, val_p, zeros)


def _mm_body(p0_ref, p1_ref, w_ref, b_ref, o_ref):
    s = p0_ref[...] + p1_ref[...]
    acc = jnp.dot(s, w_ref[...], preferred_element_type=jnp.float32)
    o_ref[...] = jnp.maximum(acc + b_ref[...], 0.0)


def _tc_finish(partial, W, b, n):
    d_in = partial.shape[2]
    d_out = W.shape[1]
    bm = 1000 if n % 1000 == 0 else n
    return pl.pallas_call(
        _mm_body,
        grid=(n // bm,),
        in_specs=[
            pl.BlockSpec((bm, d_in), lambda i: (i, 0)),
            pl.BlockSpec((bm, d_in), lambda i: (i, 0)),
            pl.BlockSpec((d_in, d_out), lambda i: (0, 0)),
            pl.BlockSpec((1, d_out), lambda i: (0, 0)),
        ],
        out_specs=pl.BlockSpec((bm, d_out), lambda i: (i, 0)),
        out_shape=jax.ShapeDtypeStruct((n, d_out), jnp.float32),
    )(partial[0], partial[1], W, b.reshape(1, d_out))


def kernel(x, adj_indices, adj_values, W, b):
    n, d = x.shape
    e = adj_values.shape[0]
    n_blocks = pl.cdiv(e, _BLK)
    b_w = pl.cdiv(n_blocks, _NW)
    pad = _NW * b_w * _BLK - e
    col_p = jnp.pad(adj_indices[1], (0, pad)).reshape(_NW, b_w, _BLK)
    row_p = jnp.pad(adj_indices[0], (0, pad)).reshape(_NW, b_w, _BLK)
    val_p = jnp.pad(adj_values, (0, pad)).reshape(_NW, b_w, _BLK)
    n_pad = ((n + 8 * _NS - 1) // (8 * _NS)) * (8 * _NS)
    zeros = jnp.zeros((n_pad // _NS, d), jnp.float32)
    partial = _sc_aggregate(x, col_p, row_p, val_p, zeros, b_w, n_pad)
    return _tc_finish(partial, W, b, n)


# speedup vs baseline: 1.5936x; 1.0608x over previous
import functools

import jax
import jax.numpy as jnp
from jax import lax
from jax.experimental import pallas as pl
from jax.experimental.pallas import tpu as pltpu
from jax.experimental.pallas import tpu_sc as plsc

_NC = 2
_NS = 16
_NW = _NC * _NS
_LANES = 16
_BLK = 128
_HB = 64


def _sc_aggregate(x, col_p, row_p, val_p, zeros, b_w, n_pad):
    _, d = x.shape
    rows_per_sub = n_pad // _NS
    mesh = plsc.VectorSubcoreMesh(core_axis_name="c", subcore_axis_name="s")

    @functools.partial(
        pl.kernel,
        out_type=jax.ShapeDtypeStruct((_NC, n_pad, d), jnp.float32),
        mesh=mesh,
        scratch_types=[
            pltpu.VMEM((b_w, _BLK), jnp.int32),
            pltpu.VMEM((b_w, _BLK), jnp.int32),
            pltpu.VMEM((b_w, _BLK), jnp.float32),
            pltpu.VMEM((_HB, d), jnp.float32),
            pltpu.VMEM((_HB, d), jnp.float32),
            pltpu.VMEM_SHARED((n_pad, d), jnp.float32),
            pltpu.SemaphoreType.DMA,
            pltpu.SemaphoreType.DMA,
            pltpu.SemaphoreType.DMA,
            pltpu.SemaphoreType.DMA,
        ],
    )
    def body(x_hbm, col_hbm, row_hbm, val_hbm, z_hbm, out_hbm,
             col_v, row_v, val_v, rows_a, rows_b, acc_sh,
             gsem_a, gsem_b, ssem_a, ssem_b):
        cid = lax.axis_index("c")
        sid = lax.axis_index("s")
        wid = sid * _NC + cid
        sub_rows = pl.ds(sid * rows_per_sub, rows_per_sub)

        pltpu.sync_copy(z_hbm, acc_sh.at[sub_rows])
        pltpu.sync_copy(col_hbm.at[wid], col_v)
        pltpu.sync_copy(row_hbm.at[wid], row_v)
        pltpu.sync_copy(val_hbm.at[wid], val_v)
        plsc.subcore_barrier()

        def scale(j, h, rows):
            def scale16(g, c2):
                vv = val_v[j, pl.ds(h * _HB + g * _LANES, _LANES)]
                base = g * _LANES
                for kk in range(_LANES):
                    v = vv[kk]
                    for c in range(d // _LANES):
                        sl = pl.ds(c * _LANES, _LANES)
                        rows[base + kk, sl] = rows[base + kk, sl] * v
                return c2

            lax.fori_loop(0, _HB // _LANES, scale16, 0)

        def process_block(j, carry):
            ga = pltpu.async_copy(
                x_hbm.at[col_v.at[j, pl.ds(0, _HB)]], rows_a, gsem_a)
            gb = pltpu.async_copy(
                x_hbm.at[col_v.at[j, pl.ds(_HB, _HB)]], rows_b, gsem_b)
            ga.wait()
            scale(j, 0, rows_a)
            sa = pltpu.async_copy(
                rows_a, acc_sh.at[row_v.at[j, pl.ds(0, _HB)]], ssem_a,
                add=True)
            gb.wait()
            scale(j, 1, rows_b)
            sb = pltpu.async_copy(
                rows_b, acc_sh.at[row_v.at[j, pl.ds(_HB, _HB)]], ssem_b,
                add=True)
            sa.wait()
            sb.wait()
            return carry

        lax.fori_loop(0, b_w, process_block, 0)

        plsc.subcore_barrier()
        pltpu.sync_copy(acc_sh.at[sub_rows], out_hbm.at[cid].at[sub_rows])

    return body(x, col_p, row_p, val_p, zeros)


def _mm_body(p0_ref, p1_ref, w_ref, b_ref, o_ref):
    s = p0_ref[...] + p1_ref[...]
    acc = jnp.dot(s, w_ref[...], preferred_element_type=jnp.float32)
    o_ref[...] = jnp.maximum(acc + b_ref[...], 0.0)


def _tc_finish(partial, W, b, n):
    d_in = partial.shape[2]
    d_out = W.shape[1]
    bm = 1000 if n % 1000 == 0 else n
    return pl.pallas_call(
        _mm_body,
        grid=(n // bm,),
        in_specs=[
            pl.BlockSpec((bm, d_in), lambda i: (i, 0)),
            pl.BlockSpec((bm, d_in), lambda i: (i, 0)),
            pl.BlockSpec((d_in, d_out), lambda i: (0, 0)),
            pl.BlockSpec((1, d_out), lambda i: (0, 0)),
        ],
        out_specs=pl.BlockSpec((bm, d_out), lambda i: (i, 0)),
        out_shape=jax.ShapeDtypeStruct((n, d_out), jnp.float32),
    )(partial[0], partial[1], W, b.reshape(1, d_out))


def kernel(x, adj_indices, adj_values, W, b):
    n, d = x.shape
    e = adj_values.shape[0]
    n_blocks = pl.cdiv(e, _BLK)
    b_w = pl.cdiv(n_blocks, _NW)
    pad = _NW * b_w * _BLK - e
    col_p = jnp.pad(adj_indices[1], (0, pad)).reshape(_NW, b_w, _BLK)
    row_p = jnp.pad(adj_indices[0], (0, pad)).reshape(_NW, b_w, _BLK)
    val_p = jnp.pad(adj_values, (0, pad)).reshape(_NW, b_w, _BLK)
    n_pad = ((n + 8 * _NS - 1) // (8 * _NS)) * (8 * _NS)
    zeros = jnp.zeros((n_pad // _NS, d), jnp.float32)
    partial = _sc_aggregate(x, col_p, row_p, val_p, zeros, b_w, n_pad)
    return _tc_finish(partial, W, b, n)
